# baseline (speedup 1.0000x reference)
import functools

import jax
import jax.numpy as jnp
from jax.experimental import pallas as pl

B, L, K = 16, 512, 64
START_IDX, END_IDX, PAD_IDX = 61, 62, 63


def _crf_body(scores_t_ref, tags_t_ref, prev_t_ref, maskf_t_ref,
              wsl_col_ref, wsl_row_ref, transition_ref,
              out_u_ref, out_l_ref):
    trans = transition_ref[:, :]
    max_t = jnp.max(trans)
    exp_ts = jnp.exp(trans - max_t)

    wsl_col = wsl_col_ref[:, :]

    a0_log = trans[START_IDX:START_IDX + 1, :] + scores_t_ref[0]
    m0 = jnp.max(a0_log, axis=1, keepdims=True)
    a = jnp.exp(a0_log - m0)
    off = m0
    m_prev = jnp.max(a, axis=1, keepdims=True)

    exp_ts_bf = exp_ts.astype(jnp.bfloat16)

    def one_step(t, state):
        a, m_prev, off, last_a, last_off = state
        s = jnp.dot(a.astype(jnp.bfloat16), exp_ts_bf,
                    preferred_element_type=jnp.float32)
        g = jnp.exp(scores_t_ref[t]) * (1.0 / m_prev)
        a_new = s * g
        off_new = off + (jnp.log(m_prev) + max_t)
        is_last = (wsl_col - 1) == t
        last_a = jnp.where(is_last, a_new, last_a)
        last_off = jnp.where(is_last, off_new, last_off)
        m_new = jnp.max(a_new, axis=1, keepdims=True)
        return a_new, m_new, off_new, last_a, last_off

    state = (a, m_prev, off, a, off)
    state = one_step(1, state)
    state = one_step(2, state)
    state = one_step(3, state)

    def body4(i, state):
        for j in range(4):
            state = one_step(4 * i + j, state)
        return state

    _, _, _, last_a, last_off = jax.lax.fori_loop(1, L // 4, body4, state)

    v = jnp.dot(last_a, exp_ts, preferred_element_type=jnp.float32)
    oh_end = (jax.lax.broadcasted_iota(jnp.int32, (B, K), 1) == END_IDX)
    picked = jnp.sum(jnp.where(oh_end, v, 0.0), axis=1, keepdims=True)
    ub = jnp.log(picked) + last_off + max_t
    out_u_ref[:, :] = jnp.sum(ub, axis=0, keepdims=True)

    tags_t = tags_t_ref[:, :]
    prev_t = prev_t_ref[:, :]
    maskf = maskf_t_ref[:, :]

    iota_k3 = jax.lax.broadcasted_iota(jnp.int32, (L, B, K), 2)
    oh_tag = (tags_t[:, :, None] == iota_k3).astype(jnp.float32)
    oh_prev = (prev_t[:, :, None] == iota_k3).astype(jnp.float32)

    oh_prev2 = jnp.reshape(oh_prev, (L * B, K))
    u = jnp.dot(oh_prev2, trans, preferred_element_type=jnp.float32)
    u3 = jnp.reshape(u, (L, B, K))
    per_elem = jnp.sum(oh_tag * (u3 + scores_t_ref[:, :, :]), axis=2)
    seq_sum = jnp.sum(per_elem * maskf)

    iota_l = jax.lax.broadcasted_iota(jnp.int32, (L, B), 0)
    is_last_t = (iota_l == (wsl_row_ref[:, :] - 1)).astype(jnp.int32)
    last_tag = jnp.sum(tags_t * is_last_t, axis=0, keepdims=True)
    iota_kb = jax.lax.broadcasted_iota(jnp.int32, (K, B), 0)
    oh_last = (last_tag == iota_kb).astype(jnp.float32)
    cnt = jnp.sum(oh_last, axis=1, keepdims=True)
    end_sum = jnp.sum(cnt * trans[:, END_IDX:END_IDX + 1])

    total = seq_sum + end_sum
    out_l_ref[:, :] = jnp.reshape(total, (1, 1))


@jax.jit
def kernel(lstm_scores, word_seq_lens, tags, mask, transition):
    scores_t = jnp.transpose(lstm_scores, (1, 0, 2))
    tags_t = jnp.transpose(tags, (1, 0))
    prev = jnp.concatenate(
        [jnp.full((B, 1), START_IDX, dtype=tags.dtype), tags[:, :-1]], axis=1)
    prev_t = jnp.transpose(prev, (1, 0))
    maskf_t = jnp.transpose(mask.astype(jnp.float32), (1, 0))
    wsl_col = word_seq_lens.reshape(B, 1)
    wsl_row = word_seq_lens.reshape(1, B)

    out_u, out_l = pl.pallas_call(
        _crf_body,
        out_shape=[
            jax.ShapeDtypeStruct((1, 1), jnp.float32),
            jax.ShapeDtypeStruct((1, 1), jnp.float32),
        ],
    )(scores_t, tags_t, prev_t, maskf_t, wsl_col, wsl_row, transition)
    return (out_u.reshape(()), out_l.reshape(()))


# speedup vs baseline: 1.0225x; 1.0225x over previous
import functools

import jax
import jax.numpy as jnp
from jax.experimental import pallas as pl

B, L, K = 16, 512, 64
START_IDX, END_IDX, PAD_IDX = 61, 62, 63


def _crf_body(scores_t_ref, tags_t_ref, prev_t_ref, maskf_t_ref,
              wsl_col_ref, wsl_row_ref, transition_ref,
              out_u_ref, out_l_ref):
    trans = transition_ref[:, :]
    max_t = jnp.max(trans)
    exp_ts = jnp.exp(trans - max_t)

    wsl_col = wsl_col_ref[:, :]

    a0_log = trans[START_IDX:START_IDX + 1, :] + scores_t_ref[0]
    m0 = jnp.max(a0_log, axis=1, keepdims=True)
    a = jnp.exp(a0_log - m0)
    off = m0
    m_prev = jnp.max(a, axis=1, keepdims=True)

    def one_step(t, state):
        a, m_prev, off, last_a, last_off = state
        s = jnp.dot(a, exp_ts, preferred_element_type=jnp.float32)
        g = jnp.exp(scores_t_ref[t]) * (1.0 / m_prev)
        a_new = s * g
        off_new = off + (jnp.log(m_prev) + max_t)
        is_last = (wsl_col - 1) == t
        last_a = jnp.where(is_last, a_new, last_a)
        last_off = jnp.where(is_last, off_new, last_off)
        m_new = jnp.max(a_new, axis=1, keepdims=True)
        return a_new, m_new, off_new, last_a, last_off

    state = (a, m_prev, off, a, off)
    state = one_step(1, state)
    state = one_step(2, state)
    state = one_step(3, state)

    def body4(i, state):
        for j in range(4):
            state = one_step(4 * i + j, state)
        return state

    _, _, _, last_a, last_off = jax.lax.fori_loop(1, L // 4, body4, state)

    v = jnp.dot(last_a, exp_ts, preferred_element_type=jnp.float32)
    oh_end = (jax.lax.broadcasted_iota(jnp.int32, (B, K), 1) == END_IDX)
    picked = jnp.sum(jnp.where(oh_end, v, 0.0), axis=1, keepdims=True)
    ub = jnp.log(picked) + last_off + max_t
    out_u_ref[:, :] = jnp.sum(ub, axis=0, keepdims=True)

    tags_t = tags_t_ref[:, :]
    prev_t = prev_t_ref[:, :]
    maskf = maskf_t_ref[:, :]

    iota_k3 = jax.lax.broadcasted_iota(jnp.int32, (L, B, K), 2)
    oh_tag = (tags_t[:, :, None] == iota_k3).astype(jnp.float32)
    oh_prev = (prev_t[:, :, None] == iota_k3).astype(jnp.float32)

    oh_prev2 = jnp.reshape(oh_prev, (L * B, K))
    u = jnp.dot(oh_prev2, trans, preferred_element_type=jnp.float32,
                precision=jax.lax.Precision.HIGHEST)
    u3 = jnp.reshape(u, (L, B, K))
    per_elem = jnp.sum(oh_tag * (u3 + scores_t_ref[:, :, :]), axis=2)
    seq_sum = jnp.sum(per_elem * maskf)

    iota_l = jax.lax.broadcasted_iota(jnp.int32, (L, B), 0)
    is_last_t = (iota_l == (wsl_row_ref[:, :] - 1)).astype(jnp.int32)
    last_tag = jnp.sum(tags_t * is_last_t, axis=0, keepdims=True)
    iota_kb = jax.lax.broadcasted_iota(jnp.int32, (K, B), 0)
    oh_last = (last_tag == iota_kb).astype(jnp.float32)
    cnt = jnp.sum(oh_last, axis=1, keepdims=True)
    end_sum = jnp.sum(cnt * trans[:, END_IDX:END_IDX + 1])

    total = seq_sum + end_sum
    out_l_ref[:, :] = jnp.reshape(total, (1, 1))


@jax.jit
def kernel(lstm_scores, word_seq_lens, tags, mask, transition):
    scores_t = jnp.transpose(lstm_scores, (1, 0, 2))
    tags_t = jnp.transpose(tags, (1, 0))
    prev = jnp.concatenate(
        [jnp.full((B, 1), START_IDX, dtype=tags.dtype), tags[:, :-1]], axis=1)
    prev_t = jnp.transpose(prev, (1, 0))
    maskf_t = jnp.transpose(mask.astype(jnp.float32), (1, 0))
    wsl_col = word_seq_lens.reshape(B, 1)
    wsl_row = word_seq_lens.reshape(1, B)

    out_u, out_l = pl.pallas_call(
        _crf_body,
        out_shape=[
            jax.ShapeDtypeStruct((1, 1), jnp.float32),
            jax.ShapeDtypeStruct((1, 1), jnp.float32),
        ],
    )(scores_t, tags_t, prev_t, maskf_t, wsl_col, wsl_row, transition)
    return (out_u.reshape(()), out_l.reshape(()))


# speedup vs baseline: 1.0580x; 1.0348x over previous
import functools

import jax
import jax.numpy as jnp
from jax.experimental import pallas as pl

B, L, K = 16, 512, 64
START_IDX, END_IDX, PAD_IDX = 61, 62, 63


def _crf_body(scores_t_ref, tags_t_ref, prev_t_ref, maskf_t_ref,
              wsl_col_ref, wsl_row_ref, transition_ref,
              out_u_ref, out_l_ref):
    trans = transition_ref[:, :]
    max_t = jnp.max(trans)
    exp_ts = jnp.exp(trans - max_t)

    wsl_col = wsl_col_ref[:, :]
    wsl_row = wsl_row_ref[:, :]

    def labeled_chunk(t0):
        tags_c = tags_t_ref[pl.ds(t0, 4), :]
        prev_c = prev_t_ref[pl.ds(t0, 4), :]
        maskf_c = maskf_t_ref[pl.ds(t0, 4), :]
        scores_c = scores_t_ref[pl.ds(t0, 4), :, :]
        iota_k3 = jax.lax.broadcasted_iota(jnp.int32, (4, B, K), 2)
        oh_tag = (tags_c[:, :, None] == iota_k3).astype(jnp.float32)
        oh_prev = (prev_c[:, :, None] == iota_k3).astype(jnp.float32)
        u = jnp.dot(jnp.reshape(oh_prev, (4 * B, K)), trans,
                    preferred_element_type=jnp.float32,
                    precision=jax.lax.Precision.HIGHEST)
        u3 = jnp.reshape(u, (4, B, K))
        contrib = jnp.sum(oh_tag * (u3 + scores_c), axis=2) * maskf_c
        iota_t = jax.lax.broadcasted_iota(jnp.int32, (4, B), 0) + t0
        is_last = (iota_t == (wsl_row - 1)).astype(jnp.int32)
        lt_part = tags_c * is_last
        return contrib, lt_part

    a0_log = trans[START_IDX:START_IDX + 1, :] + scores_t_ref[0]
    m0 = jnp.max(a0_log, axis=1, keepdims=True)
    a = jnp.exp(a0_log - m0)
    off = m0
    m_prev = jnp.max(a, axis=1, keepdims=True)

    def one_step(t, state):
        a, m_prev, off, last_a, last_off = state
        s = jnp.dot(a, exp_ts, preferred_element_type=jnp.float32)
        g = jnp.exp(scores_t_ref[t]) * (1.0 / m_prev)
        a_new = s * g
        off_new = off + (jnp.log(m_prev) + max_t)
        is_last = (wsl_col - 1) == t
        last_a = jnp.where(is_last, a_new, last_a)
        last_off = jnp.where(is_last, off_new, last_off)
        m_new = jnp.max(a_new, axis=1, keepdims=True)
        return a_new, m_new, off_new, last_a, last_off

    state = (a, m_prev, off, a, off)
    state = one_step(1, state)
    state = one_step(2, state)
    state = one_step(3, state)

    acc0, lt0 = labeled_chunk(0)

    def body4(i, carry):
        state, acc, lt = carry
        for j in range(4):
            state = one_step(4 * i + j, state)
        c, lt_part = labeled_chunk(4 * i)
        return state, acc + c, lt + lt_part

    (_, _, _, last_a, last_off), acc, lt = jax.lax.fori_loop(
        1, L // 4, body4, (state, acc0, lt0))

    v = jnp.dot(last_a, exp_ts, preferred_element_type=jnp.float32)
    oh_end = (jax.lax.broadcasted_iota(jnp.int32, (B, K), 1) == END_IDX)
    picked = jnp.sum(jnp.where(oh_end, v, 0.0), axis=1, keepdims=True)
    ub = jnp.log(picked) + last_off + max_t
    out_u_ref[:, :] = jnp.sum(ub, axis=0, keepdims=True)

    seq_sum = jnp.sum(acc)
    last_tag = jnp.sum(lt, axis=0, keepdims=True)
    iota_kb = jax.lax.broadcasted_iota(jnp.int32, (K, B), 0)
    oh_last = (last_tag == iota_kb).astype(jnp.float32)
    cnt = jnp.sum(oh_last, axis=1, keepdims=True)
    end_sum = jnp.sum(cnt * trans[:, END_IDX:END_IDX + 1])

    total = seq_sum + end_sum
    out_l_ref[:, :] = jnp.reshape(total, (1, 1))


@jax.jit
def kernel(lstm_scores, word_seq_lens, tags, mask, transition):
    scores_t = jnp.transpose(lstm_scores, (1, 0, 2))
    tags_t = jnp.transpose(tags, (1, 0))
    prev = jnp.concatenate(
        [jnp.full((B, 1), START_IDX, dtype=tags.dtype), tags[:, :-1]], axis=1)
    prev_t = jnp.transpose(prev, (1, 0))
    maskf_t = jnp.transpose(mask.astype(jnp.float32), (1, 0))
    wsl_col = word_seq_lens.reshape(B, 1)
    wsl_row = word_seq_lens.reshape(1, B)

    out_u, out_l = pl.pallas_call(
        _crf_body,
        out_shape=[
            jax.ShapeDtypeStruct((1, 1), jnp.float32),
            jax.ShapeDtypeStruct((1, 1), jnp.float32),
        ],
    )(scores_t, tags_t, prev_t, maskf_t, wsl_col, wsl_row, transition)
    return (out_u.reshape(()), out_l.reshape(()))


# speedup vs baseline: 1.1076x; 1.0469x over previous
import functools

import jax
import jax.numpy as jnp
from jax.experimental import pallas as pl

B, L, K = 16, 512, 64
START_IDX, END_IDX, PAD_IDX = 61, 62, 63


def _crf_body(scores_t_ref, tags_t_ref, prev_t_ref, maskf_t_ref,
              wsl_col_ref, wsl_row_ref, transition_ref,
              out_u_ref, out_l_ref):
    trans = transition_ref[:, :]
    max_t = jnp.max(trans)
    exp_ts = jnp.exp(trans - max_t)

    wsl_col = wsl_col_ref[:, :]
    wsl_row = wsl_row_ref[:, :]

    def labeled_chunk(t0):
        tags_c = tags_t_ref[pl.ds(t0, 4), :]
        prev_c = prev_t_ref[pl.ds(t0, 4), :]
        maskf_c = maskf_t_ref[pl.ds(t0, 4), :]
        scores_c = scores_t_ref[pl.ds(t0, 4), :, :]
        iota_k3 = jax.lax.broadcasted_iota(jnp.int32, (4, B, K), 2)
        oh_tag = (tags_c[:, :, None] == iota_k3).astype(jnp.float32)
        oh_prev = (prev_c[:, :, None] == iota_k3).astype(jnp.float32)
        u = jnp.dot(jnp.reshape(oh_prev, (4 * B, K)), trans,
                    preferred_element_type=jnp.float32,
                    precision=jax.lax.Precision.HIGHEST)
        u3 = jnp.reshape(u, (4, B, K))
        contrib = jnp.sum(oh_tag * (u3 + scores_c), axis=2) * maskf_c
        iota_t = jax.lax.broadcasted_iota(jnp.int32, (4, B), 0) + t0
        is_last = (iota_t == (wsl_row - 1)).astype(jnp.int32)
        lt_part = tags_c * is_last
        return contrib, lt_part

    a0_log = trans[START_IDX:START_IDX + 1, :] + scores_t_ref[0]
    m0 = jnp.max(a0_log, axis=1, keepdims=True)
    a = jnp.exp(a0_log - m0)
    off = m0
    m_prev = jnp.max(a, axis=1, keepdims=True)

    def one_step(t, state):
        a, m_prev, off, last_a, last_off = state
        s = jnp.dot(a, exp_ts, preferred_element_type=jnp.float32)
        g = jnp.exp(scores_t_ref[t]) * (1.0 / m_prev)
        a_new = s * g
        off_new = off + (jnp.log(m_prev) + max_t)
        is_last = (wsl_col - 1) == t
        last_a = jnp.where(is_last, a_new, last_a)
        last_off = jnp.where(is_last, off_new, last_off)
        m_new = jnp.max(a_new, axis=1, keepdims=True)
        return a_new, m_new, off_new, last_a, last_off

    state = (a, m_prev, off, a, off)
    state = one_step(1, state)
    state = one_step(2, state)
    state = one_step(3, state)

    state = one_step(4, state)
    state = one_step(5, state)
    state = one_step(6, state)
    state = one_step(7, state)

    acc0a, lt0a = labeled_chunk(0)
    acc0b, lt0b = labeled_chunk(4)
    acc0 = acc0a + acc0b
    lt0 = lt0a + lt0b

    def body8(i, carry):
        state, acc, lt = carry
        for j in range(8):
            state = one_step(8 * i + j, state)
        ca, lta = labeled_chunk(8 * i)
        cb, ltb = labeled_chunk(8 * i + 4)
        return state, acc + ca + cb, lt + lta + ltb

    (_, _, _, last_a, last_off), acc, lt = jax.lax.fori_loop(
        1, L // 8, body8, (state, acc0, lt0))

    v = jnp.dot(last_a, exp_ts, preferred_element_type=jnp.float32)
    oh_end = (jax.lax.broadcasted_iota(jnp.int32, (B, K), 1) == END_IDX)
    picked = jnp.sum(jnp.where(oh_end, v, 0.0), axis=1, keepdims=True)
    ub = jnp.log(picked) + last_off + max_t
    out_u_ref[:, :] = jnp.sum(ub, axis=0, keepdims=True)

    seq_sum = jnp.sum(acc)
    last_tag = jnp.sum(lt, axis=0, keepdims=True)
    iota_kb = jax.lax.broadcasted_iota(jnp.int32, (K, B), 0)
    oh_last = (last_tag == iota_kb).astype(jnp.float32)
    cnt = jnp.sum(oh_last, axis=1, keepdims=True)
    end_sum = jnp.sum(cnt * trans[:, END_IDX:END_IDX + 1])

    total = seq_sum + end_sum
    out_l_ref[:, :] = jnp.reshape(total, (1, 1))


@jax.jit
def kernel(lstm_scores, word_seq_lens, tags, mask, transition):
    scores_t = jnp.transpose(lstm_scores, (1, 0, 2))
    tags_t = jnp.transpose(tags, (1, 0))
    prev = jnp.concatenate(
        [jnp.full((B, 1), START_IDX, dtype=tags.dtype), tags[:, :-1]], axis=1)
    prev_t = jnp.transpose(prev, (1, 0))
    maskf_t = jnp.transpose(mask.astype(jnp.float32), (1, 0))
    wsl_col = word_seq_lens.reshape(B, 1)
    wsl_row = word_seq_lens.reshape(1, B)

    out_u, out_l = pl.pallas_call(
        _crf_body,
        out_shape=[
            jax.ShapeDtypeStruct((1, 1), jnp.float32),
            jax.ShapeDtypeStruct((1, 1), jnp.float32),
        ],
    )(scores_t, tags_t, prev_t, maskf_t, wsl_col, wsl_row, transition)
    return (out_u.reshape(()), out_l.reshape(()))


# speedup vs baseline: 1.1422x; 1.0313x over previous
import functools

import jax
import jax.numpy as jnp
from jax.experimental import pallas as pl

B, L, K = 16, 512, 64
START_IDX, END_IDX, PAD_IDX = 61, 62, 63


def _crf_body(scores_t_ref, tags_t_ref, prev_t_ref, maskf_t_ref,
              wsl_col_ref, wsl_row_ref, transition_ref,
              out_u_ref, out_l_ref):
    trans = transition_ref[:, :]
    max_t = jnp.max(trans)
    exp_ts = jnp.exp(trans - max_t)

    wsl_col = wsl_col_ref[:, :]
    wsl_row = wsl_row_ref[:, :]

    def labeled_chunk(t0):
        tags_c = tags_t_ref[pl.ds(t0, 4), :]
        prev_c = prev_t_ref[pl.ds(t0, 4), :]
        maskf_c = maskf_t_ref[pl.ds(t0, 4), :]
        scores_c = scores_t_ref[pl.ds(t0, 4), :, :]
        iota_k3 = jax.lax.broadcasted_iota(jnp.int32, (4, B, K), 2)
        oh_tag = (tags_c[:, :, None] == iota_k3).astype(jnp.float32)
        oh_prev = (prev_c[:, :, None] == iota_k3).astype(jnp.float32)
        u = jnp.dot(jnp.reshape(oh_prev, (4 * B, K)), trans,
                    preferred_element_type=jnp.float32,
                    precision=jax.lax.Precision.HIGHEST)
        u3 = jnp.reshape(u, (4, B, K))
        contrib = jnp.sum(oh_tag * (u3 + scores_c), axis=2) * maskf_c
        iota_t = jax.lax.broadcasted_iota(jnp.int32, (4, B), 0) + t0
        is_last = (iota_t == (wsl_row - 1)).astype(jnp.int32)
        lt_part = tags_c * is_last
        return contrib, lt_part

    a0_log = trans[START_IDX:START_IDX + 1, :] + scores_t_ref[0]
    m0 = jnp.max(a0_log, axis=1, keepdims=True)
    a = jnp.exp(a0_log - m0)
    off = m0
    m_prev = jnp.max(a, axis=1, keepdims=True)

    def one_step(t, state):
        a, m_prev, off, last_a, last_off = state
        s = jnp.dot(a, exp_ts, preferred_element_type=jnp.float32)
        g = jnp.exp(scores_t_ref[t]) * (1.0 / m_prev)
        a_new = s * g
        off_new = off + (jnp.log(m_prev) + max_t)
        is_last = (wsl_col - 1) == t
        last_a = jnp.where(is_last, a_new, last_a)
        last_off = jnp.where(is_last, off_new, last_off)
        m_new = jnp.max(a_new, axis=1, keepdims=True)
        return a_new, m_new, off_new, last_a, last_off

    state = (a, m_prev, off, a, off)
    state = one_step(1, state)
    state = one_step(2, state)
    state = one_step(3, state)

    for t in range(4, 16):
        state = one_step(t, state)

    acc0 = None
    lt0 = None
    for t0 in (0, 4, 8, 12):
        c, ltp = labeled_chunk(t0)
        acc0 = c if acc0 is None else acc0 + c
        lt0 = ltp if lt0 is None else lt0 + ltp

    def body16(i, carry):
        state, acc, lt = carry
        for j in range(16):
            state = one_step(16 * i + j, state)
        for j0 in (0, 4, 8, 12):
            c, ltp = labeled_chunk(16 * i + j0)
            acc = acc + c
            lt = lt + ltp
        return state, acc, lt

    (_, _, _, last_a, last_off), acc, lt = jax.lax.fori_loop(
        1, L // 16, body16, (state, acc0, lt0))

    v = jnp.dot(last_a, exp_ts, preferred_element_type=jnp.float32)
    oh_end = (jax.lax.broadcasted_iota(jnp.int32, (B, K), 1) == END_IDX)
    picked = jnp.sum(jnp.where(oh_end, v, 0.0), axis=1, keepdims=True)
    ub = jnp.log(picked) + last_off + max_t
    out_u_ref[:, :] = jnp.sum(ub, axis=0, keepdims=True)

    seq_sum = jnp.sum(acc)
    last_tag = jnp.sum(lt, axis=0, keepdims=True)
    iota_kb = jax.lax.broadcasted_iota(jnp.int32, (K, B), 0)
    oh_last = (last_tag == iota_kb).astype(jnp.float32)
    cnt = jnp.sum(oh_last, axis=1, keepdims=True)
    end_sum = jnp.sum(cnt * trans[:, END_IDX:END_IDX + 1])

    total = seq_sum + end_sum
    out_l_ref[:, :] = jnp.reshape(total, (1, 1))


@jax.jit
def kernel(lstm_scores, word_seq_lens, tags, mask, transition):
    scores_t = jnp.transpose(lstm_scores, (1, 0, 2))
    tags_t = jnp.transpose(tags, (1, 0))
    prev = jnp.concatenate(
        [jnp.full((B, 1), START_IDX, dtype=tags.dtype), tags[:, :-1]], axis=1)
    prev_t = jnp.transpose(prev, (1, 0))
    maskf_t = jnp.transpose(mask.astype(jnp.float32), (1, 0))
    wsl_col = word_seq_lens.reshape(B, 1)
    wsl_row = word_seq_lens.reshape(1, B)

    out_u, out_l = pl.pallas_call(
        _crf_body,
        out_shape=[
            jax.ShapeDtypeStruct((1, 1), jnp.float32),
            jax.ShapeDtypeStruct((1, 1), jnp.float32),
        ],
    )(scores_t, tags_t, prev_t, maskf_t, wsl_col, wsl_row, transition)
    return (out_u.reshape(()), out_l.reshape(()))
